# baseline (speedup 1.0000x reference)
import functools

import jax
import jax.numpy as jnp
from jax import lax
from jax.experimental import pallas as pl
from jax.experimental.pallas import tpu as pltpu
from jax.experimental.pallas import tpu_sc as plsc

_VOCAB = 1000000
_D = 64
_R = 16
_SCALING = 1.0

_TILE = 8192
_NSTEP = (_VOCAB + _TILE - 1) // _TILE
_NPAIR = 62
_SPLIT = _NPAIR * _TILE
_BOFF = (_NSTEP - _NPAIR) * _TILE

_B_ROWS = 16384
_SEQ = 50
_LD = _SEQ * _D
_TBLK = 256
_NTB = _B_ROWS // _TBLK


def _table_body(wta_ref, wtb_ref, aa_ref, ab_ref, b_ref, out_ref, ss_ref):
    i = pl.program_id(0)
    dta = wta_ref[...] + lax.dot_general(
        b_ref[...], aa_ref[...], (((1,), (0,)), ((), ())),
        preferred_element_type=jnp.float32) * _SCALING
    dtb = wtb_ref[...] + lax.dot_general(
        b_ref[...], ab_ref[...], (((1,), (0,)), ((), ())),
        preferred_element_type=jnp.float32) * _SCALING
    packed = jnp.concatenate([dta, dtb], axis=0).astype(jnp.bfloat16)
    out_ref[...] = packed.T
    vb = (_NSTEP - _NPAIR + i) * _TILE + lax.broadcasted_iota(
        jnp.int32, (_D, _TILE), 1)
    d2 = dta * dta + jnp.where(
        (vb >= _SPLIT) & (vb < _VOCAB), dtb * dtb, 0.0)
    part = jnp.broadcast_to(jnp.sum(d2, axis=1, keepdims=True), (_D, 8))

    @pl.when(i == 0)
    def _():
        ss_ref[...] = part

    @pl.when(i > 0)
    def _():
        ss_ref[...] += part


_NW = 32
_NTOK = _B_ROWS * _SEQ
_PER_W = _NTOK // _NW
_SUB = 128
_GPC = 8
_CHUNK = _SUB * _GPC
_NCH = _PER_W // _CHUNK


_TPC = _CHUNK // 2
_CPG = 16384 // _TPC


def _gather_body(table_hbm, xt_hbm, out_hbm, idx_v, va, vb, rows_v, sem):
    wid = lax.axis_index("s") * 2 + lax.axis_index("c")
    iota = lax.iota(jnp.int32, 16)

    def chunk(ci, carry):
        c = wid * _NCH + ci
        g = c // _CPG
        t0 = (c % _CPG) * _TPC
        pltpu.sync_copy(xt_hbm.at[2 * g, pl.ds(t0, _TPC)], va)
        pltpu.sync_copy(xt_hbm.at[2 * g + 1, pl.ds(t0, _TPC)], vb)
        for j in range(_GPC):
            for k in range(4):
                for r, buf in ((0, va), (1, vb)):
                    v = buf[pl.ds(j * 64 + k * 16, 16)]
                    u = v + v - jnp.where(v >= _SPLIT, 2 * _BOFF - 1, 0)
                    plsc.store_scatter(
                        idx_v.at[j], [2 * (k * 16 + iota) + r], u)
        copies = [
            pltpu.async_copy(table_hbm.at[idx_v.at[j]], rows_v.at[j], sem)
            for j in range(_GPC)
        ]
        for cp in copies:
            cp.wait()
        pltpu.sync_copy(rows_v, out_hbm.at[pl.ds(c * _GPC, _GPC)])
        return carry

    lax.fori_loop(0, _NCH, chunk, 0)


@functools.cache
def _make_gather():
    mesh = plsc.VectorSubcoreMesh(core_axis_name="c", subcore_axis_name="s")
    return functools.partial(
        pl.kernel,
        out_type=jax.ShapeDtypeStruct((_NTOK // _SUB, _SUB, _D), jnp.bfloat16),
        mesh=mesh,
        scratch_types=[
            pltpu.VMEM((_GPC, _SUB), jnp.int32),
            pltpu.VMEM((_TPC,), jnp.int32),
            pltpu.VMEM((_TPC,), jnp.int32),
            pltpu.VMEM((_GPC, _SUB, _D), jnp.bfloat16),
            pltpu.SemaphoreType.DMA,
        ],
        compiler_params=pltpu.CompilerParams(
            use_tc_tiling_on_sc=False, needs_layout_passes=False),
    )(_gather_body)


def _trans_body(x_ref, ss_ref, mag_ref, y_ref):
    scale_col = mag_ref[...] * jnp.sqrt(ss_ref[:, 0:1])
    sc = jnp.concatenate([scale_col] * 2, axis=0)
    y_ref[...] = x_ref[...].astype(jnp.float32).T * sc


def kernel(x, W, A, B, mag):
    Wt = W.T
    mag_col = mag.reshape(_D, 1)
    table, ss = pl.pallas_call(
        _table_body,
        grid=(_NPAIR,),
        in_specs=[
            pl.BlockSpec((_D, _TILE), lambda i: (0, i)),
            pl.BlockSpec((_D, _TILE), lambda i: (0, i + _NSTEP - _NPAIR)),
            pl.BlockSpec((_R, _TILE), lambda i: (0, i)),
            pl.BlockSpec((_R, _TILE), lambda i: (0, i + _NSTEP - _NPAIR)),
            pl.BlockSpec((_D, _R), lambda i: (0, 0)),
        ],
        out_specs=[
            pl.BlockSpec((_TILE, 2 * _D), lambda i: (i, 0)),
            pl.BlockSpec((_D, 8), lambda i: (0, 0)),
        ],
        out_shape=[
            jax.ShapeDtypeStruct((_SPLIT, 2 * _D), jnp.bfloat16),
            jax.ShapeDtypeStruct((_D, 8), jnp.float32),
        ],
    )(Wt, Wt, A, A, B)

    table64 = table.reshape(2 * _SPLIT, _D)
    xT = x.astype(jnp.int32).T
    rows = _make_gather()(table64, xT)

    xv = rows.reshape(_NTOK // 2, 128)
    y = pl.pallas_call(
        _trans_body,
        grid=(_SEQ // 2, 2),
        in_specs=[
            pl.BlockSpec((_TILE, 128), lambda g, i: (2 * g + i, 0)),
            pl.BlockSpec((_D, 8), lambda g, i: (0, 0)),
            pl.BlockSpec((_D, 1), lambda g, i: (0, 0)),
        ],
        out_specs=pl.BlockSpec((128, _TILE), lambda g, i: (g, i)),
        out_shape=jax.ShapeDtypeStruct((_LD, _B_ROWS), jnp.float32),
    )(xv, ss, mag_col)

    return jnp.transpose(y.reshape(_SEQ, _D, _B_ROWS), (2, 0, 1))


# speedup vs baseline: 2.3175x; 2.3175x over previous
import functools

import jax
import jax.numpy as jnp
from jax import lax
from jax.experimental import pallas as pl
from jax.experimental.pallas import tpu as pltpu
from jax.experimental.pallas import tpu_sc as plsc

_VOCAB = 1000000
_D = 64
_R = 16
_SCALING = 1.0

_TILE = 8192
_NSTEP = (_VOCAB + _TILE - 1) // _TILE
_NPAIR = 62
_SPLIT = _NPAIR * _TILE
_BOFF = (_NSTEP - _NPAIR) * _TILE

_B_ROWS = 16384
_SEQ = 50
_LD = _SEQ * _D
_TBLK = 256
_NTB = _B_ROWS // _TBLK


def _table_body(wta_ref, wtb_ref, aa_ref, ab_ref, b_ref, out_ref, ss_ref):
    i = pl.program_id(0)
    dta = wta_ref[...] + lax.dot_general(
        b_ref[...], aa_ref[...], (((1,), (0,)), ((), ())),
        preferred_element_type=jnp.float32) * _SCALING
    dtb = wtb_ref[...] + lax.dot_general(
        b_ref[...], ab_ref[...], (((1,), (0,)), ((), ())),
        preferred_element_type=jnp.float32) * _SCALING
    packed = jnp.concatenate([dta, dtb], axis=0)
    out_ref[...] = packed.T
    vb = (_NSTEP - _NPAIR + i) * _TILE + lax.broadcasted_iota(
        jnp.int32, (_D, _TILE), 1)
    d2 = dta * dta + jnp.where(
        (vb >= _SPLIT) & (vb < _VOCAB), dtb * dtb, 0.0)
    part = jnp.broadcast_to(jnp.sum(d2, axis=1, keepdims=True), (_D, 8))

    @pl.when(i == 0)
    def _():
        ss_ref[...] = part

    @pl.when(i > 0)
    def _():
        ss_ref[...] += part


_NW = 32
_NTOK = _B_ROWS * _SEQ
_PER_W = _NTOK // _NW
_SUB = 128
_GPC = 8
_CHUNK = _SUB * _GPC
_NCH = _PER_W // _CHUNK


_TPC = _CHUNK // 2
_CPG = 16384 // _TPC


def _gather_body(table_hbm, xt_hbm, out_hbm, idx_v, va, vb, rows_v, sem):
    wid = lax.axis_index("s") * 2 + lax.axis_index("c")
    iota = lax.iota(jnp.int32, 16)

    def chunk(ci, carry):
        c = wid * _NCH + ci
        g = c // _CPG
        t0 = (c % _CPG) * _TPC
        pltpu.sync_copy(xt_hbm.at[2 * g, pl.ds(t0, _TPC)], va)
        pltpu.sync_copy(xt_hbm.at[2 * g + 1, pl.ds(t0, _TPC)], vb)
        for j in range(_GPC):
            for k in range(4):
                for r, buf in ((0, va), (1, vb)):
                    v = buf[pl.ds(j * 64 + k * 16, 16)]
                    u = v + v - jnp.where(v >= _SPLIT, 2 * _BOFF - 1, 0)
                    plsc.store_scatter(
                        idx_v.at[j], [2 * (k * 16 + iota) + r], u)
        copies = [
            pltpu.async_copy(table_hbm.at[idx_v.at[j]], rows_v.at[j], sem)
            for j in range(_GPC)
        ]
        for cp in copies:
            cp.wait()
        pltpu.sync_copy(rows_v, out_hbm.at[pl.ds(c * _GPC, _GPC)])
        return carry

    lax.fori_loop(0, _NCH, chunk, 0)


@functools.cache
def _make_gather():
    mesh = plsc.VectorSubcoreMesh(core_axis_name="c", subcore_axis_name="s")
    return functools.partial(
        pl.kernel,
        out_type=jax.ShapeDtypeStruct((_NTOK // _SUB, _SUB, _D), jnp.float32),
        mesh=mesh,
        scratch_types=[
            pltpu.VMEM((_GPC, _SUB), jnp.int32),
            pltpu.VMEM((_TPC,), jnp.int32),
            pltpu.VMEM((_TPC,), jnp.int32),
            pltpu.VMEM((_GPC, _SUB, _D), jnp.float32),
            pltpu.SemaphoreType.DMA,
        ],
        compiler_params=pltpu.CompilerParams(
            use_tc_tiling_on_sc=False, needs_layout_passes=False),
    )(_gather_body)


def _trans_body(x_ref, ss_ref, mag_ref, y_ref):
    scale_col = mag_ref[...] * jnp.sqrt(ss_ref[:, 0:1])
    sc = jnp.concatenate([scale_col] * 2, axis=0)
    y_ref[...] = x_ref[...].T * sc


def kernel(x, W, A, B, mag):
    Wt = W.T
    mag_col = mag.reshape(_D, 1)
    table, ss = pl.pallas_call(
        _table_body,
        grid=(_NPAIR,),
        in_specs=[
            pl.BlockSpec((_D, _TILE), lambda i: (0, i)),
            pl.BlockSpec((_D, _TILE), lambda i: (0, i + _NSTEP - _NPAIR)),
            pl.BlockSpec((_R, _TILE), lambda i: (0, i)),
            pl.BlockSpec((_R, _TILE), lambda i: (0, i + _NSTEP - _NPAIR)),
            pl.BlockSpec((_D, _R), lambda i: (0, 0)),
        ],
        out_specs=[
            pl.BlockSpec((_TILE, 2 * _D), lambda i: (i, 0)),
            pl.BlockSpec((_D, 8), lambda i: (0, 0)),
        ],
        out_shape=[
            jax.ShapeDtypeStruct((_SPLIT, 2 * _D), jnp.float32),
            jax.ShapeDtypeStruct((_D, 8), jnp.float32),
        ],
    )(Wt, Wt, A, A, B)

    table64 = table.reshape(2 * _SPLIT, _D)
    xT = x.astype(jnp.int32).T
    rows = _make_gather()(table64, xT)

    xv = rows.reshape(_NTOK // 2, 128)
    y = pl.pallas_call(
        _trans_body,
        grid=(_SEQ // 2, 2),
        in_specs=[
            pl.BlockSpec((_TILE, 128), lambda g, i: (2 * g + i, 0)),
            pl.BlockSpec((_D, 8), lambda g, i: (0, 0)),
            pl.BlockSpec((_D, 1), lambda g, i: (0, 0)),
        ],
        out_specs=pl.BlockSpec((128, _TILE), lambda g, i: (g, i)),
        out_shape=jax.ShapeDtypeStruct((_LD, _B_ROWS), jnp.float32),
    )(xv, ss, mag_col)

    return jnp.transpose(y.reshape(_SEQ, _D, _B_ROWS), (2, 0, 1))


# speedup vs baseline: 2.4419x; 1.0537x over previous
import functools

import jax
import jax.numpy as jnp
from jax import lax
from jax.experimental import pallas as pl
from jax.experimental.pallas import tpu as pltpu
from jax.experimental.pallas import tpu_sc as plsc

_VOCAB = 1000000
_D = 64
_R = 16
_SCALING = 1.0

_TILE = 8192
_NSTEP = (_VOCAB + _TILE - 1) // _TILE
_NPAIR = 62
_SPLIT = _NPAIR * _TILE
_BOFF = (_NSTEP - _NPAIR) * _TILE

_B_ROWS = 16384
_SEQ = 50
_LD = _SEQ * _D
_TBLK = 256
_NTB = _B_ROWS // _TBLK


def _table_body(wta_ref, wtb_ref, aa_ref, ab_ref, b_ref, out_ref, ss_ref):
    i = pl.program_id(0)
    dta = wta_ref[...] + lax.dot_general(
        b_ref[...], aa_ref[...], (((1,), (0,)), ((), ())),
        preferred_element_type=jnp.float32) * _SCALING
    dtb = wtb_ref[...] + lax.dot_general(
        b_ref[...], ab_ref[...], (((1,), (0,)), ((), ())),
        preferred_element_type=jnp.float32) * _SCALING
    packed = jnp.concatenate([dta, dtb], axis=0)
    out_ref[...] = packed.T
    vb = (_NSTEP - _NPAIR + i) * _TILE + lax.broadcasted_iota(
        jnp.int32, (_D, _TILE), 1)
    d2 = dta * dta + jnp.where(
        (vb >= _SPLIT) & (vb < _VOCAB), dtb * dtb, 0.0)
    part = jnp.broadcast_to(jnp.sum(d2, axis=1, keepdims=True), (_D, 8))

    @pl.when(i == 0)
    def _():
        ss_ref[...] = part

    @pl.when(i > 0)
    def _():
        ss_ref[...] += part


_NW = 32
_NTOK = _B_ROWS * _SEQ
_PER_W = _NTOK // _NW
_SUB = 128
_GPC = 8
_CHUNK = _SUB * _GPC
_NCH = _PER_W // _CHUNK


_TPC = _CHUNK // 2
_CPG = 16384 // _TPC


def _gather_body(table_hbm, xt_hbm, out_hbm, idx_v, va, vb, rows_v, sem,
                 wsem):
    wid = lax.axis_index("s") * 2 + lax.axis_index("c")
    iota = lax.iota(jnp.int32, 16)

    def chunk(ci, carry):
        c = wid * _NCH + ci
        g = c // _CPG
        t0 = (c % _CPG) * _TPC
        pltpu.sync_copy(xt_hbm.at[2 * g, pl.ds(t0, _TPC)], va)
        pltpu.sync_copy(xt_hbm.at[2 * g + 1, pl.ds(t0, _TPC)], vb)
        for j in range(_GPC):
            for k in range(4):
                for r, buf in ((0, va), (1, vb)):
                    v = buf[pl.ds(j * 64 + k * 16, 16)]
                    u = v + v - jnp.where(v >= _SPLIT, 2 * _BOFF - 1, 0)
                    plsc.store_scatter(
                        idx_v.at[j], [2 * (k * 16 + iota) + r], u)
        @pl.when(ci > 0)
        def _():
            pltpu.make_async_copy(
                rows_v, out_hbm.at[pl.ds((c - 1) * _GPC, _GPC)], wsem).wait()

        copies = [
            pltpu.async_copy(table_hbm.at[idx_v.at[j]], rows_v.at[j], sem)
            for j in range(_GPC)
        ]
        for cp in copies:
            cp.wait()
        pltpu.async_copy(rows_v, out_hbm.at[pl.ds(c * _GPC, _GPC)], wsem)
        return carry

    lax.fori_loop(0, _NCH, chunk, 0)
    c_last = wid * _NCH + _NCH - 1
    pltpu.make_async_copy(
        rows_v, out_hbm.at[pl.ds(c_last * _GPC, _GPC)], wsem).wait()


@functools.cache
def _make_gather():
    mesh = plsc.VectorSubcoreMesh(core_axis_name="c", subcore_axis_name="s")
    return functools.partial(
        pl.kernel,
        out_type=jax.ShapeDtypeStruct((_NTOK // _SUB, _SUB, _D), jnp.float32),
        mesh=mesh,
        scratch_types=[
            pltpu.VMEM((_GPC, _SUB), jnp.int32),
            pltpu.VMEM((_TPC,), jnp.int32),
            pltpu.VMEM((_TPC,), jnp.int32),
            pltpu.VMEM((_GPC, _SUB, _D), jnp.float32),
            pltpu.SemaphoreType.DMA,
            pltpu.SemaphoreType.DMA,
        ],
        compiler_params=pltpu.CompilerParams(
            use_tc_tiling_on_sc=False, needs_layout_passes=False),
    )(_gather_body)


def _trans_body(x_ref, ss_ref, mag_ref, y_ref):
    scale_col = mag_ref[...] * jnp.sqrt(ss_ref[:, 0:1])
    sc = jnp.concatenate([scale_col] * 2, axis=0)
    y_ref[...] = x_ref[...].T * sc


def kernel(x, W, A, B, mag):
    Wt = W.T
    mag_col = mag.reshape(_D, 1)
    table, ss = pl.pallas_call(
        _table_body,
        grid=(_NPAIR,),
        in_specs=[
            pl.BlockSpec((_D, _TILE), lambda i: (0, i)),
            pl.BlockSpec((_D, _TILE), lambda i: (0, i + _NSTEP - _NPAIR)),
            pl.BlockSpec((_R, _TILE), lambda i: (0, i)),
            pl.BlockSpec((_R, _TILE), lambda i: (0, i + _NSTEP - _NPAIR)),
            pl.BlockSpec((_D, _R), lambda i: (0, 0)),
        ],
        out_specs=[
            pl.BlockSpec((_TILE, 2 * _D), lambda i: (i, 0)),
            pl.BlockSpec((_D, 8), lambda i: (0, 0)),
        ],
        out_shape=[
            jax.ShapeDtypeStruct((_SPLIT, 2 * _D), jnp.float32),
            jax.ShapeDtypeStruct((_D, 8), jnp.float32),
        ],
    )(Wt, Wt, A, A, B)

    table64 = table.reshape(2 * _SPLIT, _D)
    xT = x.astype(jnp.int32).T
    rows = _make_gather()(table64, xT)

    xv = rows.reshape(_NTOK // 2, 128)
    y = pl.pallas_call(
        _trans_body,
        grid=(_SEQ // 2, 2),
        in_specs=[
            pl.BlockSpec((_TILE, 128), lambda g, i: (2 * g + i, 0)),
            pl.BlockSpec((_D, 8), lambda g, i: (0, 0)),
            pl.BlockSpec((_D, 1), lambda g, i: (0, 0)),
        ],
        out_specs=pl.BlockSpec((128, _TILE), lambda g, i: (g, i)),
        out_shape=jax.ShapeDtypeStruct((_LD, _B_ROWS), jnp.float32),
    )(xv, ss, mag_col)

    return jnp.transpose(y.reshape(_SEQ, _D, _B_ROWS), (2, 0, 1))
